# baseline (speedup 1.0000x reference)
import functools

import jax
import jax.numpy as jnp
from jax.experimental import pallas as pl

_B, _N, _F, _K = 32, 8192, 16, 1024
_EPS = 1e-10
_IMIN = -2147483648


def _min_kernel(x_ref, o_ref):
    b = pl.program_id(0)
    m = jnp.full((1, 1), jnp.min(jnp.abs(x_ref[0])), jnp.float32)

    @pl.when(b == 0)
    def _():
        o_ref[:, :] = m

    @pl.when(b != 0)
    def _():
        o_ref[:, :] = jnp.minimum(o_ref[:, :], m)


def _select_kernel(xt_ref, m0t_ref, minv_ref, out_ref, mask_ref):
    x = xt_ref[0]
    m0 = m0t_ref[0]
    v = (x + (minv_ref[:, :] + _EPS)) * m0
    bits = jax.lax.bitcast_convert_type(v, jnp.int32)
    keys = jnp.where(bits < 0, bits ^ jnp.int32(0x7FFFFFFF), bits)

    khi = (keys >> 16).astype(jnp.int16)
    klo = ((keys & 0xFFFF) - 32768).astype(jnp.int16)

    ones = jnp.ones((_N, 1), jnp.bfloat16)
    kf = jnp.float32(_K)

    def _count(cmp):
        sel = jnp.where(cmp, jnp.bfloat16(1), jnp.bfloat16(0))
        return jax.lax.dot_general(sel, ones, (((1,), (0,)), ((), ())),
                                   preferred_element_type=jnp.float32)

    def hbody(i, p):
        cand = p | jax.lax.shift_left(jnp.int32(1), 31 - i)
        ch = ((cand ^ jnp.int32(_IMIN)) >> 16).astype(jnp.int16)
        return jnp.where(_count(khi >= ch) >= kf, cand, p)

    p = jax.lax.fori_loop(0, 16, hbody, jnp.zeros((_F, 1), jnp.int32))
    h16 = ((p ^ jnp.int32(_IMIN)) >> 16).astype(jnp.int16)
    band_v = (khi == h16).astype(jnp.int16)
    ghi = _count(khi > h16)
    klo_m = jnp.where(band_v == 1, klo, jnp.int16(-32768))

    def lbody(i, p):
        cand = p | jax.lax.shift_left(jnp.int32(1), 15 - i)
        cl = ((cand & 0xFFFF) - 32768).astype(jnp.int16)
        return jnp.where(ghi + _count(klo_m >= cl) >= kf, cand, p)

    p = jax.lax.fori_loop(0, 16, lbody, p)
    tlo = ((p & 0xFFFF) - 32768).astype(jnp.int16)

    gt_v = (khi > h16).astype(jnp.int16) | (klo_m > tlo).astype(jnp.int16)
    eq_v = band_v & (klo == tlo).astype(jnp.int16)
    g0 = _count(gt_v == 1)
    ne = _count(eq_v == 1)
    iota = jax.lax.broadcasted_iota(jnp.int16, (_F, _N), 1)
    iota_m = jnp.where(eq_v == 1, iota, jnp.int16(32767))

    def tie_search(_):
        def ibody(i, p2):
            cand = p2 | jax.lax.shift_left(jnp.int32(1), 12 - i)
            cnt = g0 + _count(iota_m <= cand.astype(jnp.int16))
            return jnp.where(cnt < kf, cand, p2)

        p2 = jax.lax.fori_loop(0, 13, ibody, jnp.zeros((_F, 1), jnp.int32))
        gp = g0 + _count(iota_m <= p2.astype(jnp.int16))
        return p2 + (gp < kf).astype(jnp.int32)

    no_ties = jnp.all((g0 + ne) == kf)
    jstar = jax.lax.cond(no_ties,
                         lambda _: jnp.full((_F, 1), _N - 1, jnp.int32),
                         tie_search, 0)

    sel_v = gt_v | (iota_m <= jstar.astype(jnp.int16)).astype(jnp.int16)
    maskf = jnp.max(sel_v.astype(jnp.float32), axis=0, keepdims=True)
    mask_ref[0] = maskf
    out_ref[0] = x * maskf


@jax.jit
def kernel(input, mask, init_mask):
    del mask
    xt = jnp.transpose(input, (0, 2, 1))
    m0t = jnp.transpose(init_mask, (0, 2, 1))

    minv = pl.pallas_call(
        _min_kernel,
        grid=(_B,),
        in_specs=[pl.BlockSpec((1, _F, _N), lambda b: (b, 0, 0))],
        out_specs=pl.BlockSpec((1, 1), lambda b: (0, 0)),
        out_shape=jax.ShapeDtypeStruct((1, 1), jnp.float32),
    )(xt)

    out_t, mask_t = pl.pallas_call(
        _select_kernel,
        grid=(_B,),
        in_specs=[
            pl.BlockSpec((1, _F, _N), lambda b: (b, 0, 0)),
            pl.BlockSpec((1, 1, _N), lambda b: (b, 0, 0)),
            pl.BlockSpec((1, 1), lambda b: (0, 0)),
        ],
        out_specs=[
            pl.BlockSpec((1, _F, _N), lambda b: (b, 0, 0)),
            pl.BlockSpec((1, 1, _N), lambda b: (b, 0, 0)),
        ],
        out_shape=[
            jax.ShapeDtypeStruct((_B, _F, _N), jnp.float32),
            jax.ShapeDtypeStruct((_B, 1, _N), jnp.float32),
        ],
    )(xt, m0t, minv)

    updated_mask = jnp.reshape(mask_t, (_B, _N, 1))
    masked_out = jnp.transpose(out_t, (0, 2, 1))
    return (updated_mask, masked_out)


# speedup vs baseline: 5.7387x; 5.7387x over previous
import jax
import jax.numpy as jnp
from jax.experimental import pallas as pl

_B, _N, _F, _K = 32, 8192, 16, 1024
_BB = 4
_EPS = 1e-10
_IMIN = -2147483648


def _min_kernel(x_ref, o_ref):
    b = pl.program_id(0)
    m = jnp.full((1, 1), jnp.min(jnp.abs(x_ref[...])), jnp.float32)

    @pl.when(b == 0)
    def _():
        o_ref[:, :] = m

    @pl.when(b != 0)
    def _():
        o_ref[:, :] = jnp.minimum(o_ref[:, :], m)


def _select_kernel(xt_ref, m0t_ref, minv_ref, out_ref, mask_ref):
    x = xt_ref[...]
    m0 = m0t_ref[...]
    v = (x + (minv_ref[:, :] + _EPS)[:, :, None]) * m0
    bits = jax.lax.bitcast_convert_type(v, jnp.int32)
    keys = jnp.where(bits < 0, bits ^ jnp.int32(0x7FFFFFFF), bits)

    def vbody(i, p):
        cand = p | jax.lax.shift_left(jnp.int32(1), 31 - i)
        cnt = jnp.sum((keys >= (cand ^ jnp.int32(_IMIN))).astype(jnp.int32),
                      axis=2, keepdims=True)
        return jnp.where(cnt >= _K, cand, p)

    p = jax.lax.fori_loop(0, 32, vbody, jnp.zeros((_BB, _F, 1), jnp.int32))
    tkey = p ^ jnp.int32(_IMIN)

    gt = keys > tkey
    eq = keys == tkey
    g0 = jnp.sum(gt.astype(jnp.int32), axis=2, keepdims=True)
    ne = jnp.sum(eq.astype(jnp.int32), axis=2, keepdims=True)
    iota = jax.lax.broadcasted_iota(jnp.int32, (_BB, _F, _N), 2)
    iota_m = jnp.where(eq, iota, jnp.int32(_N))

    def tie_search(_):
        def ibody(i, p2):
            cand = p2 | jax.lax.shift_left(jnp.int32(1), 12 - i)
            cnt = g0 + jnp.sum((iota_m <= cand).astype(jnp.int32), axis=2,
                               keepdims=True)
            return jnp.where(cnt < _K, cand, p2)

        p2 = jax.lax.fori_loop(0, 13, ibody,
                               jnp.zeros((_BB, _F, 1), jnp.int32))
        gp = g0 + jnp.sum((iota_m <= p2).astype(jnp.int32), axis=2,
                          keepdims=True)
        return p2 + (gp < _K).astype(jnp.int32)

    no_ties = jnp.all((g0 + ne) == _K)
    jstar = jax.lax.cond(no_ties,
                         lambda _: jnp.full((_BB, _F, 1), _N - 1, jnp.int32),
                         tie_search, 0)

    sel = gt | (iota_m <= jstar)
    maskf = jnp.max(sel.astype(jnp.float32), axis=1, keepdims=True)
    mask_ref[...] = maskf
    out_ref[...] = x * maskf


@jax.jit
def kernel(input, mask, init_mask):
    del mask
    xt = jnp.transpose(input, (0, 2, 1))
    m0t = jnp.transpose(init_mask, (0, 2, 1))

    minv = pl.pallas_call(
        _min_kernel,
        grid=(_B // _BB,),
        in_specs=[pl.BlockSpec((_BB, _F, _N), lambda b: (b, 0, 0))],
        out_specs=pl.BlockSpec((1, 1), lambda b: (0, 0)),
        out_shape=jax.ShapeDtypeStruct((1, 1), jnp.float32),
    )(xt)

    out_t, mask_t = pl.pallas_call(
        _select_kernel,
        grid=(_B // _BB,),
        in_specs=[
            pl.BlockSpec((_BB, _F, _N), lambda b: (b, 0, 0)),
            pl.BlockSpec((_BB, 1, _N), lambda b: (b, 0, 0)),
            pl.BlockSpec((1, 1), lambda b: (0, 0)),
        ],
        out_specs=[
            pl.BlockSpec((_BB, _F, _N), lambda b: (b, 0, 0)),
            pl.BlockSpec((_BB, 1, _N), lambda b: (b, 0, 0)),
        ],
        out_shape=[
            jax.ShapeDtypeStruct((_B, _F, _N), jnp.float32),
            jax.ShapeDtypeStruct((_B, 1, _N), jnp.float32),
        ],
    )(xt, m0t, minv)

    updated_mask = jnp.reshape(mask_t, (_B, _N, 1))
    masked_out = jnp.transpose(out_t, (0, 2, 1))
    return (updated_mask, masked_out)


# speedup vs baseline: 6.9448x; 1.2102x over previous
import jax
import jax.numpy as jnp
from jax.experimental import pallas as pl

_B, _N, _F, _K = 32, 8192, 16, 1024
_BB = 8
_EPS = 1e-10
_IMIN = -2147483648


def _min_kernel(x_ref, o_ref):
    b = pl.program_id(0)
    m = jnp.full((1, 1), jnp.min(jnp.abs(x_ref[...])), jnp.float32)

    @pl.when(b == 0)
    def _():
        o_ref[:, :] = m

    @pl.when(b != 0)
    def _():
        o_ref[:, :] = jnp.minimum(o_ref[:, :], m)


def _select_kernel(xt_ref, m0t_ref, minv_ref, out_ref, mask_ref):
    x = xt_ref[...]
    m0 = m0t_ref[...]
    v = (x + (minv_ref[:, :] + _EPS)[:, :, None]) * m0
    bits = jax.lax.bitcast_convert_type(v, jnp.int32)
    keys = jnp.where(bits < 0, bits ^ jnp.int32(0x7FFFFFFF), bits)

    def vbody(i, pc):
        p, c = pc
        cand = p | jax.lax.shift_left(jnp.int32(1), 31 - i)
        cnt = jnp.sum((keys >= (cand ^ jnp.int32(_IMIN))).astype(jnp.int32),
                      axis=2, keepdims=True)
        acc = cnt >= _K
        return jnp.where(acc, cand, p), jnp.where(acc, cnt, c)

    p, c = jax.lax.fori_loop(
        0, 32, vbody, (jnp.zeros((_BB, _F, 1), jnp.int32),
                       jnp.full((_BB, _F, 1), _N, jnp.int32)))
    tkey = p ^ jnp.int32(_IMIN)

    gt = keys > tkey
    eq = keys == tkey
    iota = jax.lax.broadcasted_iota(jnp.int32, (_BB, _F, _N), 2)
    iota_m = jnp.where(eq, iota, jnp.int32(_N))

    def tie_search(_):
        g0 = jnp.sum(gt.astype(jnp.int32), axis=2, keepdims=True)

        def ibody(i, p2):
            cand = p2 | jax.lax.shift_left(jnp.int32(1), 12 - i)
            cnt = g0 + jnp.sum((iota_m <= cand).astype(jnp.int32), axis=2,
                               keepdims=True)
            return jnp.where(cnt < _K, cand, p2)

        p2 = jax.lax.fori_loop(0, 13, ibody,
                               jnp.zeros((_BB, _F, 1), jnp.int32))
        gp = g0 + jnp.sum((iota_m <= p2).astype(jnp.int32), axis=2,
                          keepdims=True)
        return p2 + (gp < _K).astype(jnp.int32)

    no_ties = jnp.all(c == _K)
    jstar = jax.lax.cond(no_ties,
                         lambda _: jnp.full((_BB, _F, 1), _N - 1, jnp.int32),
                         tie_search, 0)

    sel = gt | (iota_m <= jstar)
    maskf = jnp.max(sel.astype(jnp.float32), axis=1, keepdims=True)
    mask_ref[...] = maskf
    out_ref[...] = x * maskf


@jax.jit
def kernel(input, mask, init_mask):
    del mask
    xt = jnp.transpose(input, (0, 2, 1))
    m0t = jnp.transpose(init_mask, (0, 2, 1))

    minv = pl.pallas_call(
        _min_kernel,
        grid=(_B // _BB,),
        in_specs=[pl.BlockSpec((_BB, _F, _N), lambda b: (b, 0, 0))],
        out_specs=pl.BlockSpec((1, 1), lambda b: (0, 0)),
        out_shape=jax.ShapeDtypeStruct((1, 1), jnp.float32),
    )(xt)

    out_t, mask_t = pl.pallas_call(
        _select_kernel,
        grid=(_B // _BB,),
        in_specs=[
            pl.BlockSpec((_BB, _F, _N), lambda b: (b, 0, 0)),
            pl.BlockSpec((_BB, 1, _N), lambda b: (b, 0, 0)),
            pl.BlockSpec((1, 1), lambda b: (0, 0)),
        ],
        out_specs=[
            pl.BlockSpec((_BB, _F, _N), lambda b: (b, 0, 0)),
            pl.BlockSpec((_BB, 1, _N), lambda b: (b, 0, 0)),
        ],
        out_shape=[
            jax.ShapeDtypeStruct((_B, _F, _N), jnp.float32),
            jax.ShapeDtypeStruct((_B, 1, _N), jnp.float32),
        ],
    )(xt, m0t, minv)

    updated_mask = jnp.reshape(mask_t, (_B, _N, 1))
    masked_out = jnp.transpose(out_t, (0, 2, 1))
    return (updated_mask, masked_out)


# speedup vs baseline: 7.1114x; 1.0240x over previous
import jax
import jax.numpy as jnp
from jax.experimental import pallas as pl

_B, _N, _F, _K = 32, 8192, 16, 1024
_BB = 8
_EPS = 1e-10
_IMIN = -2147483648


def _min_kernel(x_ref, o_ref):
    b = pl.program_id(0)
    m = jnp.full((1, 1), jnp.min(jnp.abs(x_ref[...])), jnp.float32)

    @pl.when(b == 0)
    def _():
        o_ref[:, :] = m

    @pl.when(b != 0)
    def _():
        o_ref[:, :] = jnp.minimum(o_ref[:, :], m)


def _select_kernel(xt_ref, m0t_ref, minv_ref, out_ref, mask_ref):
    x = xt_ref[...]
    m0 = m0t_ref[...]
    v = (x + (minv_ref[:, :] + _EPS)[:, :, None]) * m0
    bits = jax.lax.bitcast_convert_type(v, jnp.int32)
    keys = jnp.where(bits < 0, bits ^ jnp.int32(0x7FFFFFFF), bits)

    def vbody(i, pc):
        p, c = pc
        cand = p | jax.lax.shift_left(jnp.int32(1), 31 - i)
        cnt = jnp.sum((keys >= (cand ^ jnp.int32(_IMIN))).astype(jnp.int32),
                      axis=2, keepdims=True)
        acc = cnt >= _K
        return jnp.where(acc, cand, p), jnp.where(acc, cnt, c)

    p, c = jax.lax.fori_loop(
        0, 32, vbody, (jnp.zeros((_BB, _F, 1), jnp.int32),
                       jnp.full((_BB, _F, 1), _N, jnp.int32)))
    tkey = p ^ jnp.int32(_IMIN)

    no_ties = jnp.all(c == _K)

    @pl.when(no_ties)
    def _():
        sel = keys >= tkey
        maskf = jnp.max(sel.astype(jnp.float32), axis=1, keepdims=True)
        mask_ref[...] = maskf
        out_ref[...] = x * maskf

    @pl.when(jnp.logical_not(no_ties))
    def _():
        gt = keys > tkey
        eq = keys == tkey
        iota = jax.lax.broadcasted_iota(jnp.int32, (_BB, _F, _N), 2)
        iota_m = jnp.where(eq, iota, jnp.int32(_N))
        g0 = jnp.sum(gt.astype(jnp.int32), axis=2, keepdims=True)

        def ibody(i, p2):
            cand = p2 | jax.lax.shift_left(jnp.int32(1), 12 - i)
            cnt = g0 + jnp.sum((iota_m <= cand).astype(jnp.int32), axis=2,
                               keepdims=True)
            return jnp.where(cnt < _K, cand, p2)

        p2 = jax.lax.fori_loop(0, 13, ibody,
                               jnp.zeros((_BB, _F, 1), jnp.int32))
        gp = g0 + jnp.sum((iota_m <= p2).astype(jnp.int32), axis=2,
                          keepdims=True)
        jstar = p2 + (gp < _K).astype(jnp.int32)

        sel = gt | (iota_m <= jstar)
        maskf = jnp.max(sel.astype(jnp.float32), axis=1, keepdims=True)
        mask_ref[...] = maskf
        out_ref[...] = x * maskf


@jax.jit
def kernel(input, mask, init_mask):
    del mask
    xt = jnp.transpose(input, (0, 2, 1))
    m0t = jnp.transpose(init_mask, (0, 2, 1))

    minv = pl.pallas_call(
        _min_kernel,
        grid=(_B // _BB,),
        in_specs=[pl.BlockSpec((_BB, _F, _N), lambda b: (b, 0, 0))],
        out_specs=pl.BlockSpec((1, 1), lambda b: (0, 0)),
        out_shape=jax.ShapeDtypeStruct((1, 1), jnp.float32),
    )(xt)

    out_t, mask_t = pl.pallas_call(
        _select_kernel,
        grid=(_B // _BB,),
        in_specs=[
            pl.BlockSpec((_BB, _F, _N), lambda b: (b, 0, 0)),
            pl.BlockSpec((_BB, 1, _N), lambda b: (b, 0, 0)),
            pl.BlockSpec((1, 1), lambda b: (0, 0)),
        ],
        out_specs=[
            pl.BlockSpec((_BB, _F, _N), lambda b: (b, 0, 0)),
            pl.BlockSpec((_BB, 1, _N), lambda b: (b, 0, 0)),
        ],
        out_shape=[
            jax.ShapeDtypeStruct((_B, _F, _N), jnp.float32),
            jax.ShapeDtypeStruct((_B, 1, _N), jnp.float32),
        ],
    )(xt, m0t, minv)

    updated_mask = jnp.reshape(mask_t, (_B, _N, 1))
    masked_out = jnp.transpose(out_t, (0, 2, 1))
    return (updated_mask, masked_out)


# speedup vs baseline: 7.7699x; 1.0926x over previous
import jax
import jax.numpy as jnp
from jax.experimental import pallas as pl

_B, _N, _F, _K = 32, 8192, 16, 1024
_BB = 8
_EPS = 1e-10
_IMIN = -2147483648


def _min_kernel(x_ref, o_ref):
    b = pl.program_id(0)
    m = jnp.full((1, 1), jnp.min(jnp.abs(x_ref[...])), jnp.float32)

    @pl.when(b == 0)
    def _():
        o_ref[:, :] = m

    @pl.when(b != 0)
    def _():
        o_ref[:, :] = jnp.minimum(o_ref[:, :], m)


def _select_kernel(xt_ref, m0t_ref, minv_ref, out_ref, mask_ref):
    x = xt_ref[...]
    m0 = m0t_ref[...]
    v = (x + (minv_ref[:, :] + _EPS)[:, :, None]) * m0
    bits = jax.lax.bitcast_convert_type(v, jnp.int32)
    keys = jnp.where(bits < 0, bits ^ jnp.int32(0x7FFFFFFF), bits)

    def vcond(ipc):
        i, _, c = ipc
        return jnp.logical_and(i < 32, jnp.logical_not(jnp.all(c == _K)))

    def vbody(ipc):
        i, p, c = ipc
        cand = p | jax.lax.shift_left(jnp.int32(1), 31 - i)
        cnt = jnp.sum((keys >= (cand ^ jnp.int32(_IMIN))).astype(jnp.int32),
                      axis=2, keepdims=True)
        acc = cnt >= _K
        return i + 1, jnp.where(acc, cand, p), jnp.where(acc, cnt, c)

    _, p, c = jax.lax.while_loop(
        vcond, vbody, (jnp.int32(0), jnp.zeros((_BB, _F, 1), jnp.int32),
                       jnp.full((_BB, _F, 1), _N, jnp.int32)))
    tkey = p ^ jnp.int32(_IMIN)

    no_ties = jnp.all(c == _K)

    @pl.when(no_ties)
    def _():
        sel = keys >= tkey
        maskf = jnp.max(sel.astype(jnp.float32), axis=1, keepdims=True)
        mask_ref[...] = maskf
        out_ref[...] = x * maskf

    @pl.when(jnp.logical_not(no_ties))
    def _():
        gt = keys > tkey
        eq = keys == tkey
        iota = jax.lax.broadcasted_iota(jnp.int32, (_BB, _F, _N), 2)
        iota_m = jnp.where(eq, iota, jnp.int32(_N))
        g0 = jnp.sum(gt.astype(jnp.int32), axis=2, keepdims=True)

        def ibody(i, p2):
            cand = p2 | jax.lax.shift_left(jnp.int32(1), 12 - i)
            cnt = g0 + jnp.sum((iota_m <= cand).astype(jnp.int32), axis=2,
                               keepdims=True)
            return jnp.where(cnt < _K, cand, p2)

        p2 = jax.lax.fori_loop(0, 13, ibody,
                               jnp.zeros((_BB, _F, 1), jnp.int32))
        gp = g0 + jnp.sum((iota_m <= p2).astype(jnp.int32), axis=2,
                          keepdims=True)
        jstar = p2 + (gp < _K).astype(jnp.int32)

        sel = gt | (iota_m <= jstar)
        maskf = jnp.max(sel.astype(jnp.float32), axis=1, keepdims=True)
        mask_ref[...] = maskf
        out_ref[...] = x * maskf


@jax.jit
def kernel(input, mask, init_mask):
    del mask
    xt = jnp.transpose(input, (0, 2, 1))
    m0t = jnp.transpose(init_mask, (0, 2, 1))

    minv = pl.pallas_call(
        _min_kernel,
        grid=(_B // _BB,),
        in_specs=[pl.BlockSpec((_BB, _F, _N), lambda b: (b, 0, 0))],
        out_specs=pl.BlockSpec((1, 1), lambda b: (0, 0)),
        out_shape=jax.ShapeDtypeStruct((1, 1), jnp.float32),
    )(xt)

    out_t, mask_t = pl.pallas_call(
        _select_kernel,
        grid=(_B // _BB,),
        in_specs=[
            pl.BlockSpec((_BB, _F, _N), lambda b: (b, 0, 0)),
            pl.BlockSpec((_BB, 1, _N), lambda b: (b, 0, 0)),
            pl.BlockSpec((1, 1), lambda b: (0, 0)),
        ],
        out_specs=[
            pl.BlockSpec((_BB, _F, _N), lambda b: (b, 0, 0)),
            pl.BlockSpec((_BB, 1, _N), lambda b: (b, 0, 0)),
        ],
        out_shape=[
            jax.ShapeDtypeStruct((_B, _F, _N), jnp.float32),
            jax.ShapeDtypeStruct((_B, 1, _N), jnp.float32),
        ],
    )(xt, m0t, minv)

    updated_mask = jnp.reshape(mask_t, (_B, _N, 1))
    masked_out = jnp.transpose(out_t, (0, 2, 1))
    return (updated_mask, masked_out)
